# baseline (speedup 1.0000x reference)
import functools

import jax
import jax.numpy as jnp
from jax import lax
from jax.experimental import pallas as pl
from jax.experimental.pallas import tpu as pltpu
from jax.experimental.pallas import tpu_sc as plsc

B, T, D, H = 16, 4096, 128, 512
_NCHUNK = T // 16


def _mlp_kernel(x_ref, w1_ref, b1_ref, w2_ref, b2_ref, w3_ref, b3_ref, out_ref):
    wc = lax.dot_general(w3_ref[...], w2_ref[...], (((1,), (0,)), ((), ())),
                         preferred_element_type=jnp.float32)
    c = jnp.sum(w3_ref[...] * b2_ref[...], axis=1, keepdims=True) + b3_ref[...]
    wcb = wc.astype(jnp.bfloat16)
    w1 = w1_ref[...].astype(jnp.bfloat16)
    b1b = b1_ref[...].astype(jnp.bfloat16)

    x = x_ref[0].astype(jnp.bfloat16)
    h32 = lax.dot_general(x, w1, (((1,), (1,)), ((), ())),
                          preferred_element_type=jnp.float32)
    h = jnp.maximum(h32.astype(jnp.bfloat16) + b1b, jnp.bfloat16(0.0))
    z = lax.dot_general(wcb, h, (((1,), (1,)), ((), ())),
                        preferred_element_type=jnp.float32)
    out_ref[0] = jax.nn.sigmoid(z + c)


def _tc_topk_kernel(logits_ref, len_ref, out_ref):
    nb = logits_ref.shape[0]
    v_raw = logits_ref[...]
    L = len_ref[...]
    k = L // 16 + 1
    col = lax.broadcasted_iota(jnp.int32, (nb, T), 1)
    v = jnp.where(col < L, v_raw, -1.0)
    lo0 = jnp.zeros((nb, 1), jnp.int32)
    hi0 = jnp.full((nb, 1), 0x3F800000, jnp.int32)

    def body(_, carry):
        lo, hi = carry
        mid = (lo + hi) >> 1
        t = lax.bitcast_convert_type(mid, jnp.float32)
        cnt = jnp.sum((v >= t).astype(jnp.int32), axis=1, keepdims=True)
        ge = cnt >= k
        return jnp.where(ge, mid, lo), jnp.where(ge, hi, mid)

    lo, _ = lax.fori_loop(0, 31, body, (lo0, hi0))
    xk = lax.bitcast_convert_type(lo, jnp.float32)
    gt = v > xk
    cnt_gt = jnp.sum(gt.astype(jnp.int32), axis=1, keepdims=True)
    sum_gt = jnp.sum(jnp.where(gt, v, 0.0), axis=1, keepdims=True)
    kf = k.astype(jnp.float32)
    out_ref[...] = (sum_gt + (k - cnt_gt).astype(jnp.float32) * xk) / kf


_UNR = 16


def _sc_topk_kernel(logits_hbm, lens_hbm, out_hbm, row_v, lens_v, res_v,
                    buf_f, buf_i, sem):
    cid = lax.axis_index("c")
    sid = lax.axis_index("s")
    w = sid * 2 + cid

    @pl.when(w < B // 2)
    def _():
        pltpu.sync_copy(logits_hbm.at[w], row_v)
        pltpu.sync_copy(lens_hbm, lens_v)
        lane = lax.iota(jnp.int32, 16)
        one = jnp.ones((16,), jnp.int32)
        zero = jnp.zeros((16,), jnp.int32)

        def rotsum_f(acc):
            for sh in (1, 2, 4, 8):
                buf_f[pl.ds(0, 16)] = acc
                buf_f[pl.ds(16, 16)] = acc
                acc = acc + buf_f[pl.ds(sh, 16)]
            return acc

        def rotsum_i(acc):
            for sh in (1, 2, 4, 8):
                buf_i[pl.ds(0, 16)] = acc
                buf_i[pl.ds(16, 16)] = acc
                acc = acc + buf_i[pl.ds(sh, 16)]
            return acc

        Lv = rotsum_i(jnp.where(lane == w, lens_v[...], 0))
        kv = (Lv >> 4) + 1

        def mask_body(jj, _):
            base = jj * (16 * _UNR)
            for u in range(_UNR):
                pos = lane + (base + u * 16)
                v = row_v[pl.ds(base + u * 16, 16)]
                row_v[pl.ds(base + u * 16, 16)] = jnp.where(pos < Lv, v, -1.0)
            return 0

        lax.fori_loop(0, _NCHUNK // _UNR, mask_body, 0)

        def bis(_, carry):
            lo, hi = carry
            mid = (lo + hi) >> 1
            t = lax.bitcast_convert_type(mid, jnp.float32)

            def cnt_body(jj, cnts):
                base = jj * (16 * _UNR)
                cnts = list(cnts)
                for u in range(_UNR):
                    cnts[u % 4] = cnts[u % 4] + jnp.where(
                        row_v[pl.ds(base + u * 16, 16)] >= t, one, zero)
                return tuple(cnts)

            c0, c1, c2, c3 = lax.fori_loop(0, _NCHUNK // _UNR, cnt_body,
                                           (zero, zero, zero, zero))
            cnt = (c0 + c1) + (c2 + c3)
            ge = rotsum_i(cnt) >= kv
            return jnp.where(ge, mid, lo), jnp.where(ge, hi, mid)

        lo0 = jnp.zeros((16,), jnp.int32)
        hi0 = jnp.full((16,), 0x3F800000, jnp.int32)
        lo, _ = lax.fori_loop(0, 31, bis, (lo0, hi0))
        xk = lax.bitcast_convert_type(lo, jnp.float32)

        def fin_body(jj, carry):
            cacc, sacc = carry
            base = jj * (16 * _UNR)
            for u in range(_UNR):
                v = row_v[pl.ds(base + u * 16, 16)]
                gt = v > xk
                cacc = cacc + jnp.where(gt, one, zero)
                sacc = sacc + jnp.where(gt, v, 0.0)
            return cacc, sacc

        cacc, sacc = lax.fori_loop(0, _NCHUNK // _UNR, fin_body,
                                   (zero, jnp.zeros((16,), jnp.float32)))
        cnt_gt = rotsum_i(cacc)
        sum_gt = rotsum_f(sacc)
        kf = kv.astype(jnp.float32)
        res_v[...] = (sum_gt + (kv - cnt_gt).astype(jnp.float32) * xk) / kf
        pltpu.sync_copy(res_v, out_hbm.at[w])


def _sc_topk_half(logits, lens):
    mesh = plsc.VectorSubcoreMesh(core_axis_name="c", subcore_axis_name="s")
    fn = functools.partial(
        pl.kernel,
        mesh=mesh,
        out_type=jax.ShapeDtypeStruct((B // 2, 16), jnp.float32),
        scratch_types=[
            pltpu.VMEM((T,), jnp.float32),
            pltpu.VMEM((16,), jnp.int32),
            pltpu.VMEM((16,), jnp.float32),
            pltpu.VMEM((32,), jnp.float32),
            pltpu.VMEM((32,), jnp.int32),
            pltpu.SemaphoreType.DMA,
        ],
    )(_sc_topk_kernel)
    return fn(logits, lens)


def kernel(avf_out, seq_len, W1, b1, W2, b2, W3, b3):
    b1r = b1.reshape(1, H)
    b2r = b2.reshape(1, 32)
    b3r = b3.reshape(1, 1)
    lens = seq_len.astype(jnp.int32).reshape(B)
    HB = B // 2

    def mlp_half(base):
        out2 = pl.pallas_call(
            _mlp_kernel,
            grid=(HB,),
            in_specs=[
                pl.BlockSpec((1, T, D), lambda i: (i + base, 0, 0)),
                pl.BlockSpec((H, D), lambda i: (0, 0)),
                pl.BlockSpec((1, H), lambda i: (0, 0)),
                pl.BlockSpec((32, H), lambda i: (0, 0)),
                pl.BlockSpec((1, 32), lambda i: (0, 0)),
                pl.BlockSpec((1, 32), lambda i: (0, 0)),
                pl.BlockSpec((1, 1), lambda i: (0, 0)),
            ],
            out_specs=pl.BlockSpec((1, 1, T), lambda i: (i, 0, 0)),
            out_shape=jax.ShapeDtypeStruct((HB, 1, T), jnp.float32),
        )(avf_out, W1, b1r, W2, b2r, W3, b3r)
        return out2.reshape(HB, T)

    logitsA = mlp_half(0)
    lensA = jnp.concatenate([lens[:HB], jnp.full((B - HB,), T, jnp.int32)])
    resA = _sc_topk_half(logitsA, lensA)

    logitsB = mlp_half(HB)
    resB = pl.pallas_call(
        _tc_topk_kernel,
        out_shape=jax.ShapeDtypeStruct((HB, 1), jnp.float32),
    )(logitsB, lens[HB:].reshape(HB, 1))

    return jnp.concatenate([resA[:, 0], resB[:, 0]])


# speedup vs baseline: 1.0651x; 1.0651x over previous
import functools

import jax
import jax.numpy as jnp
from jax import lax
from jax.experimental import pallas as pl
from jax.experimental.pallas import tpu as pltpu
from jax.experimental.pallas import tpu_sc as plsc

B, T, D, H = 16, 4096, 128, 512
_NCHUNK = T // 16


def _mlp_kernel(x_ref, w1_ref, b1_ref, w2_ref, b2_ref, w3_ref, b3_ref, out_ref):
    wc = lax.dot_general(w3_ref[...], w2_ref[...], (((1,), (0,)), ((), ())),
                         preferred_element_type=jnp.float32)
    c = jnp.sum(w3_ref[...] * b2_ref[...], axis=1, keepdims=True) + b3_ref[...]
    x = x_ref[0]
    h = lax.dot_general(x, w1_ref[...], (((1,), (1,)), ((), ())),
                        preferred_element_type=jnp.float32,
                        precision=lax.Precision.DEFAULT)
    h = jnp.maximum(h + b1_ref[...], 0.0)
    z = lax.dot_general(wc, h, (((1,), (1,)), ((), ())),
                        preferred_element_type=jnp.float32,
                        precision=lax.Precision.DEFAULT)
    out_ref[0] = jax.nn.sigmoid(z + c)


_UNR = 16


def _sc_topk_kernel(logits_hbm, lens_hbm, out_hbm, row_v, lens_v, res_v,
                    buf_f, buf_i, sem):
    cid = lax.axis_index("c")
    sid = lax.axis_index("s")
    w = sid * 2 + cid

    @pl.when(w < B)
    def _():
        pltpu.sync_copy(logits_hbm.at[w], row_v)
        pltpu.sync_copy(lens_hbm, lens_v)
        lane = lax.iota(jnp.int32, 16)
        one = jnp.ones((16,), jnp.int32)
        zero = jnp.zeros((16,), jnp.int32)

        def rotsum_f(acc):
            for sh in (1, 2, 4, 8):
                buf_f[pl.ds(0, 16)] = acc
                buf_f[pl.ds(16, 16)] = acc
                acc = acc + buf_f[pl.ds(sh, 16)]
            return acc

        def rotsum_i(acc):
            for sh in (1, 2, 4, 8):
                buf_i[pl.ds(0, 16)] = acc
                buf_i[pl.ds(16, 16)] = acc
                acc = acc + buf_i[pl.ds(sh, 16)]
            return acc

        Lv = rotsum_i(jnp.where(lane == w, lens_v[...], 0))
        kv = (Lv >> 4) + 1

        def mask_body(jj, _):
            base = jj * (16 * _UNR)
            for u in range(_UNR):
                pos = lane + (base + u * 16)
                v = row_v[pl.ds(base + u * 16, 16)]
                row_v[pl.ds(base + u * 16, 16)] = jnp.where(pos < Lv, v, -1.0)
            return 0

        lax.fori_loop(0, _NCHUNK // _UNR, mask_body, 0)

        def bis(_, carry):
            lo, hi = carry
            mid = (lo + hi) >> 1
            t = lax.bitcast_convert_type(mid, jnp.float32)

            def cnt_body(jj, cnts):
                base = jj * (16 * _UNR)
                cnts = list(cnts)
                for u in range(_UNR):
                    cnts[u % 4] = cnts[u % 4] + jnp.where(
                        row_v[pl.ds(base + u * 16, 16)] >= t, one, zero)
                return tuple(cnts)

            c0, c1, c2, c3 = lax.fori_loop(0, _NCHUNK // _UNR, cnt_body,
                                           (zero, zero, zero, zero))
            cnt = (c0 + c1) + (c2 + c3)
            ge = rotsum_i(cnt) >= kv
            return jnp.where(ge, mid, lo), jnp.where(ge, hi, mid)

        lo0 = jnp.zeros((16,), jnp.int32)
        hi0 = jnp.full((16,), 0x3F800000, jnp.int32)
        lo, _ = lax.fori_loop(0, 31, bis, (lo0, hi0))
        xk = lax.bitcast_convert_type(lo, jnp.float32)

        def fin_body(jj, carry):
            cacc, sacc = carry
            base = jj * (16 * _UNR)
            for u in range(_UNR):
                v = row_v[pl.ds(base + u * 16, 16)]
                gt = v > xk
                cacc = cacc + jnp.where(gt, one, zero)
                sacc = sacc + jnp.where(gt, v, 0.0)
            return cacc, sacc

        cacc, sacc = lax.fori_loop(0, _NCHUNK // _UNR, fin_body,
                                   (zero, jnp.zeros((16,), jnp.float32)))
        cnt_gt = rotsum_i(cacc)
        sum_gt = rotsum_f(sacc)
        kf = kv.astype(jnp.float32)
        res_v[...] = (sum_gt + (kv - cnt_gt).astype(jnp.float32) * xk) / kf
        pltpu.sync_copy(res_v, out_hbm.at[w])


def _sc_topk(logits, lens):
    mesh = plsc.VectorSubcoreMesh(core_axis_name="c", subcore_axis_name="s")
    fn = functools.partial(
        pl.kernel,
        mesh=mesh,
        out_type=jax.ShapeDtypeStruct((B, 16), jnp.float32),
        scratch_types=[
            pltpu.VMEM((T,), jnp.float32),
            pltpu.VMEM((16,), jnp.int32),
            pltpu.VMEM((16,), jnp.float32),
            pltpu.VMEM((32,), jnp.float32),
            pltpu.VMEM((32,), jnp.int32),
            pltpu.SemaphoreType.DMA,
        ],
    )(_sc_topk_kernel)
    return fn(logits, lens)


def kernel(avf_out, seq_len, W1, b1, W2, b2, W3, b3):
    b1r = b1.reshape(1, H)
    b2r = b2.reshape(1, 32)
    b3r = b3.reshape(1, 1)
    lens = seq_len.astype(jnp.int32).reshape(B)

    logits3 = pl.pallas_call(
        _mlp_kernel,
        grid=(B,),
        in_specs=[
            pl.BlockSpec((1, T, D), lambda i: (i, 0, 0)),
            pl.BlockSpec((H, D), lambda i: (0, 0)),
            pl.BlockSpec((1, H), lambda i: (0, 0)),
            pl.BlockSpec((32, H), lambda i: (0, 0)),
            pl.BlockSpec((1, 32), lambda i: (0, 0)),
            pl.BlockSpec((1, 32), lambda i: (0, 0)),
            pl.BlockSpec((1, 1), lambda i: (0, 0)),
        ],
        out_specs=pl.BlockSpec((1, 1, T), lambda i: (i, 0, 0)),
        out_shape=jax.ShapeDtypeStruct((B, 1, T), jnp.float32),
    )(avf_out, W1, b1r, W2, b2r, W3, b3r)
    logits = logits3.reshape(B, T)

    res = _sc_topk(logits, lens)
    return res[:, 0]


# speedup vs baseline: 1.0945x; 1.0276x over previous
import functools

import jax
import jax.numpy as jnp
from jax import lax
from jax.experimental import pallas as pl
from jax.experimental.pallas import tpu as pltpu
from jax.experimental.pallas import tpu_sc as plsc

B, T, D, H = 16, 4096, 128, 512
_NCHUNK = T // 16


def _mlp_kernel(x_ref, w1_ref, b1_ref, w2_ref, b2_ref, w3_ref, b3_ref, out_ref):
    wc = lax.dot_general(w3_ref[...], w2_ref[...], (((1,), (0,)), ((), ())),
                         preferred_element_type=jnp.float32)
    c = jnp.sum(w3_ref[...] * b2_ref[...], axis=1, keepdims=True) + b3_ref[...]
    x = x_ref[0]
    h = lax.dot_general(x, w1_ref[...], (((1,), (1,)), ((), ())),
                        preferred_element_type=jnp.float32,
                        precision=lax.Precision.DEFAULT)
    h = jnp.maximum(h + b1_ref[...], 0.0)
    z = lax.dot_general(wc, h, (((1,), (1,)), ((), ())),
                        preferred_element_type=jnp.float32,
                        precision=lax.Precision.DEFAULT)
    out_ref[...] = jax.nn.sigmoid(z + c)[0]


_UNR = 16


def _sc_topk_kernel(logits_hbm, lens_hbm, out_hbm, row_v, lens_v, res_v,
                    buf_f, buf_i, sem):
    cid = lax.axis_index("c")
    sid = lax.axis_index("s")
    w = sid * 2 + cid

    @pl.when(w < B)
    def _():
        pltpu.sync_copy(logits_hbm.at[pl.ds(w * T, T)], row_v)
        pltpu.sync_copy(lens_hbm, lens_v)
        lane = lax.iota(jnp.int32, 16)
        one = jnp.ones((16,), jnp.int32)
        zero = jnp.zeros((16,), jnp.int32)

        def rotsum_f(acc):
            for sh in (1, 2, 4, 8):
                buf_f[pl.ds(0, 16)] = acc
                buf_f[pl.ds(16, 16)] = acc
                acc = acc + buf_f[pl.ds(sh, 16)]
            return acc

        def rotsum_i(acc):
            for sh in (1, 2, 4, 8):
                buf_i[pl.ds(0, 16)] = acc
                buf_i[pl.ds(16, 16)] = acc
                acc = acc + buf_i[pl.ds(sh, 16)]
            return acc

        Lv = rotsum_i(jnp.where(lane == w, lens_v[...], 0))
        kv = (Lv >> 4) + 1

        def mask_body(jj, _):
            base = jj * (16 * _UNR)
            for u in range(_UNR):
                pos = lane + (base + u * 16)
                v = row_v[pl.ds(base + u * 16, 16)]
                row_v[pl.ds(base + u * 16, 16)] = jnp.where(pos < Lv, v, -1.0)
            return 0

        lax.fori_loop(0, _NCHUNK // _UNR, mask_body, 0)

        def bis(_, carry):
            lo, hi = carry
            mid = (lo + hi) >> 1
            t = lax.bitcast_convert_type(mid, jnp.float32)

            def cnt_body(jj, cnts):
                base = jj * (16 * _UNR)
                cnts = list(cnts)
                for u in range(_UNR):
                    cnts[u % 4] = cnts[u % 4] + jnp.where(
                        row_v[pl.ds(base + u * 16, 16)] >= t, one, zero)
                return tuple(cnts)

            c0, c1, c2, c3 = lax.fori_loop(0, _NCHUNK // _UNR, cnt_body,
                                           (zero, zero, zero, zero))
            cnt = (c0 + c1) + (c2 + c3)
            ge = rotsum_i(cnt) >= kv
            return jnp.where(ge, mid, lo), jnp.where(ge, hi, mid)

        lo0 = jnp.zeros((16,), jnp.int32)
        hi0 = jnp.full((16,), 0x3F800000, jnp.int32)
        lo, _ = lax.fori_loop(0, 31, bis, (lo0, hi0))
        xk = lax.bitcast_convert_type(lo, jnp.float32)

        def fin_body(jj, carry):
            cacc, sacc = carry
            base = jj * (16 * _UNR)
            for u in range(_UNR):
                v = row_v[pl.ds(base + u * 16, 16)]
                gt = v > xk
                cacc = cacc + jnp.where(gt, one, zero)
                sacc = sacc + jnp.where(gt, v, 0.0)
            return cacc, sacc

        cacc, sacc = lax.fori_loop(0, _NCHUNK // _UNR, fin_body,
                                   (zero, jnp.zeros((16,), jnp.float32)))
        cnt_gt = rotsum_i(cacc)
        sum_gt = rotsum_f(sacc)
        kf = kv.astype(jnp.float32)
        res_v[...] = (sum_gt + (kv - cnt_gt).astype(jnp.float32) * xk) / kf
        pltpu.sync_copy(res_v, out_hbm.at[w])


def _sc_topk(logits, lens):
    mesh = plsc.VectorSubcoreMesh(core_axis_name="c", subcore_axis_name="s")
    fn = functools.partial(
        pl.kernel,
        mesh=mesh,
        out_type=jax.ShapeDtypeStruct((B, 16), jnp.float32),
        scratch_types=[
            pltpu.VMEM((T,), jnp.float32),
            pltpu.VMEM((16,), jnp.int32),
            pltpu.VMEM((16,), jnp.float32),
            pltpu.VMEM((32,), jnp.float32),
            pltpu.VMEM((32,), jnp.int32),
            pltpu.SemaphoreType.DMA,
        ],
    )(_sc_topk_kernel)
    return fn(logits, lens)


def kernel(avf_out, seq_len, W1, b1, W2, b2, W3, b3):
    b1r = b1.reshape(1, H)
    b2r = b2.reshape(1, 32)
    b3r = b3.reshape(1, 1)
    lens = seq_len.astype(jnp.int32).reshape(B)

    logits3 = pl.pallas_call(
        _mlp_kernel,
        grid=(B,),
        in_specs=[
            pl.BlockSpec((1, T, D), lambda i: (i, 0, 0)),
            pl.BlockSpec((H, D), lambda i: (0, 0)),
            pl.BlockSpec((1, H), lambda i: (0, 0)),
            pl.BlockSpec((32, H), lambda i: (0, 0)),
            pl.BlockSpec((1, 32), lambda i: (0, 0)),
            pl.BlockSpec((1, 32), lambda i: (0, 0)),
            pl.BlockSpec((1, 1), lambda i: (0, 0)),
        ],
        out_specs=pl.BlockSpec((T,), lambda i: (i,)),
        out_shape=jax.ShapeDtypeStruct((B * T,), jnp.float32),
    )(avf_out, W1, b1r, W2, b2r, W3, b3r)
    logits = logits3

    res = _sc_topk(logits, lens)
    return res[:, 0]
